# Initial kernel scaffold; baseline (speedup 1.0000x reference)
import jax
import jax.numpy as jnp
from jax.experimental import pallas as pl


def kernel(nodes, lin_W0, lin_b0, bn_g0, bn_b0, gat_W0, att_src0, att_dst0, gat_b0, lin_W1, lin_b1, bn_g1, bn_b1, gat_W1, att_src1, att_dst1, gat_b1, edges):
    raise NotImplementedError("write your pallas kernel here")


# speedup vs baseline: 26.7044x; 26.7044x over previous
import functools

import jax
import jax.numpy as jnp
from jax import lax
from jax.experimental import pallas as pl
from jax.experimental.pallas import tpu as pltpu
from jax.experimental.pallas import tpu_sc as plsc

N = 10000
E = 320000
F = 128
HID = 128
NC = 64
H = 4

NSUB = 16
NCORE = 2
NW = NSUB * NCORE
NPAD = 10112
ROWS_PER_SUB = NPAD // NSUB

C = 64
EPW_RAW = E // NW
NCHUNK = -(-EPW_RAW // C)
EPW = NCHUNK * C

_PREC = jax.lax.Precision.HIGHEST


def _dot(a, b):
    return jax.lax.dot_general(a, b, (((1,), (0,)), ((), ())),
                               precision=_PREC,
                               preferred_element_type=jnp.float32)


def _dense_tail(x, gw_ref, asrc_ref, adst_ref, h_ref, esd_ref, hid):
    h = _dot(x, gw_ref[...])
    ys = h * asrc_ref[...]
    yd = h * adst_ref[...]
    k = hid // H
    cols = [jnp.sum(ys[:, i * k:(i + 1) * k], axis=1, keepdims=True)
            for i in range(H)]
    cols += [jnp.sum(yd[:, i * k:(i + 1) * k], axis=1, keepdims=True)
             for i in range(H)]
    cols.append(jnp.zeros((N, 16 - 2 * H), jnp.float32))
    esd = jnp.concatenate(cols, axis=1)
    zpad = jnp.zeros((NPAD - N, hid), jnp.float32)
    h_ref[...] = jnp.concatenate([h, zpad], axis=0)
    esd_ref[...] = jnp.concatenate([esd, jnp.zeros((NPAD - N, 16), jnp.float32)],
                                   axis=0)


def _dense_core(x, w_ref, b_ref, g_ref, bb_ref):
    x = _dot(x, w_ref[...]) + b_ref[...]
    m = jnp.mean(x, axis=0, keepdims=True)
    xc = x - m
    v = jnp.mean(xc * xc, axis=0, keepdims=True)
    x = xc / jnp.sqrt(v + 1e-5) * g_ref[...] + bb_ref[...]
    return jnp.maximum(x, 0.01 * x)


def _dense0_body(nodes_ref, w_ref, b_ref, g_ref, bb_ref, gw_ref, asrc_ref,
                 adst_ref, h_ref, esd_ref):
    x = _dense_core(nodes_ref[...], w_ref, b_ref, g_ref, bb_ref)
    _dense_tail(x, gw_ref, asrc_ref, adst_ref, h_ref, esd_ref, HID)


def _agg_from_partials(p_ref, hid):
    acc = p_ref[0] + p_ref[1]
    k = hid // H
    xs = []
    for i in range(H):
        den = acc[:N, hid + i:hid + i + 1] + 1e-16
        xs.append(acc[:N, i * k:(i + 1) * k] / den)
    return jnp.concatenate(xs, axis=1)


def _agg0_body(p_ref, gb0_ref, x_ref):
    x_ref[...] = _agg_from_partials(p_ref, HID) + gb0_ref[...]


def _dense1_body(x_ref, w_ref, b_ref, g_ref, bb_ref, gw_ref, asrc_ref,
                 adst_ref, h_ref, esd_ref):
    x = _dense_core(x_ref[...], w_ref, b_ref, g_ref, bb_ref)
    _dense_tail(x, gw_ref, asrc_ref, adst_ref, h_ref, esd_ref, NC)


def _fin_body(p_ref, gb1_ref, out_ref):
    o = _agg_from_partials(p_ref, NC) + gb1_ref[...]
    m = jnp.max(o, axis=1, keepdims=True)
    o = o - m
    lse = jnp.log(jnp.sum(jnp.exp(o), axis=1, keepdims=True))
    out_ref[...] = o - lse


_TC_PARAMS = pltpu.CompilerParams(vmem_limit_bytes=50 * 1024 * 1024)

_dense0_call = pl.pallas_call(
    _dense0_body,
    compiler_params=_TC_PARAMS,
    out_shape=[jax.ShapeDtypeStruct((NPAD, HID), jnp.float32),
               jax.ShapeDtypeStruct((NPAD, 16), jnp.float32)])

_agg0_call = pl.pallas_call(
    _agg0_body,
    compiler_params=_TC_PARAMS,
    out_shape=jax.ShapeDtypeStruct((N, HID), jnp.float32))

_dense1_call = pl.pallas_call(
    _dense1_body,
    compiler_params=_TC_PARAMS,
    out_shape=[jax.ShapeDtypeStruct((NPAD, NC), jnp.float32),
               jax.ShapeDtypeStruct((NPAD, 16), jnp.float32)])

_fin_call = pl.pallas_call(
    _fin_body,
    compiler_params=_TC_PARAMS,
    out_shape=jax.ShapeDtypeStruct((N, NC), jnp.float32))


def _make_sc_edge(hid):
    w = hid + 16
    nv = hid // 16
    vk = nv // H
    mesh = plsc.VectorSubcoreMesh(core_axis_name="c", subcore_axis_name="s",
                                  num_cores=NCORE, num_subcores=NSUB)

    @functools.partial(
        pl.kernel,
        out_type=jax.ShapeDtypeStruct((NCORE * NPAD, w), jnp.float32),
        mesh=mesh,
        compiler_params=pltpu.CompilerParams(needs_layout_passes=False,
                                             use_tc_tiling_on_sc=False),
        scratch_types=[
            pltpu.VMEM((C,), jnp.int32),
            pltpu.VMEM((C,), jnp.int32),
            pltpu.VMEM((C, 16), jnp.float32),
            pltpu.VMEM((C, 16), jnp.float32),
            pltpu.VMEM((C, hid), jnp.float32),
            pltpu.VMEM((C, w), jnp.float32),
            pltpu.VMEM((C, 16), jnp.float32),
            pltpu.VMEM_SHARED((NPAD, w), jnp.float32),
        ],
    )
    def sc_edge(h_hbm, esd_hbm, src_hbm, dst_hbm, z_hbm, part_hbm,
                src_buf, dst_buf, esd_s, esd_d, hrows, upd, wbuf, acc):
        cid = lax.axis_index("c")
        sid = lax.axis_index("s")
        wid = sid * NCORE + cid

        def _zrow(r, carry):
            wbuf[r, :] = jnp.zeros((16,), jnp.float32)
            return carry
        lax.fori_loop(0, C, _zrow, 0)

        r0 = sid * ROWS_PER_SUB
        pltpu.sync_copy(z_hbm.at[pl.ds(r0, ROWS_PER_SUB)],
                        acc.at[pl.ds(r0, ROWS_PER_SUB)])
        plsc.subcore_barrier()

        ebase = wid * EPW
        lanes = lax.iota(jnp.int32, 16)

        def _chunk(kk, carry):
            cb = ebase + kk * C
            pltpu.sync_copy(src_hbm.at[pl.ds(cb, C)], src_buf)
            pltpu.sync_copy(dst_hbm.at[pl.ds(cb, C)], dst_buf)
            pltpu.sync_copy(esd_hbm.at[src_buf], esd_s)
            pltpu.sync_copy(esd_hbm.at[dst_buf], esd_d)
            pltpu.sync_copy(h_hbm.at[src_buf], hrows)

            for g in range(C // 16):
                ridx = lanes + g * 16
                for head in range(H):
                    cidx = jnp.full((16,), head, jnp.int32)
                    es = plsc.load_gather(esd_s, [ridx, cidx])
                    ed = plsc.load_gather(esd_d, [ridx, cidx + H])
                    s = es + ed
                    wv = jnp.exp(jnp.maximum(s, 0.2 * s))
                    plsc.store_scatter(wbuf, [ridx, cidx], wv)

            def _edge(c, carry):
                wrow = wbuf[c, :]
                upd[c, pl.ds(hid, 16)] = wrow
                for head in range(H):
                    ws = jnp.full((16,), wrow[head])
                    for j in range(vk):
                        off = (head * vk + j) * 16
                        upd[c, pl.ds(off, 16)] = hrows[c, pl.ds(off, 16)] * ws
                return carry
            lax.fori_loop(0, C, _edge, 0)

            pltpu.sync_copy(upd, acc.at[dst_buf], add=True)
            return carry

        lax.fori_loop(0, NCHUNK, _chunk, 0)
        plsc.subcore_barrier()
        pltpu.sync_copy(acc.at[pl.ds(r0, ROWS_PER_SUB)],
                        part_hbm.at[pl.ds(cid * NPAD + r0, ROWS_PER_SUB)])

    return sc_edge


_sc_cache = {}


def _sc_edge(hid):
    if hid not in _sc_cache:
        _sc_cache[hid] = _make_sc_edge(hid)
    return _sc_cache[hid]


def kernel(nodes, lin_W0, lin_b0, bn_g0, bn_b0, gat_W0, att_src0, att_dst0,
           gat_b0, lin_W1, lin_b1, bn_g1, bn_b1, gat_W1, att_src1, att_dst1,
           gat_b1, edges):
    f32 = jnp.float32
    row = lambda a: a.reshape(1, -1).astype(f32)

    src = edges[:, 0].astype(jnp.int32).reshape(NW, EPW_RAW)
    dst = edges[:, 1].astype(jnp.int32).reshape(NW, EPW_RAW)
    pad = N + (jnp.arange(EPW - EPW_RAW, dtype=jnp.int32) % (NPAD - N))
    pad = jnp.broadcast_to(pad, (NW, EPW - EPW_RAW))
    src_p = jnp.concatenate([src, pad], axis=1).reshape(-1)
    dst_p = jnp.concatenate([dst, pad], axis=1).reshape(-1)

    z0 = jnp.zeros((NPAD, HID + 16), f32)
    z1 = jnp.zeros((NPAD, NC + 16), f32)

    h0, esd0 = _dense0_call(nodes.astype(f32), lin_W0, row(lin_b0),
                            row(bn_g0), row(bn_b0), gat_W0,
                            row(att_src0), row(att_dst0))
    p0 = _sc_edge(HID)(h0, esd0, src_p, dst_p, z0).reshape(NCORE, NPAD, HID + 16)
    x1 = _agg0_call(p0, row(gat_b0))
    h1, esd1 = _dense1_call(x1, lin_W1, row(lin_b1),
                            row(bn_g1), row(bn_b1), gat_W1,
                            row(att_src1), row(att_dst1))
    p1 = _sc_edge(NC)(h1, esd1, src_p, dst_p, z1).reshape(NCORE, NPAD, NC + 16)
    return _fin_call(p1, row(gat_b1))
